# baseline (speedup 1.0000x reference)
import functools

import jax
import jax.numpy as jnp
from jax import lax
from jax.experimental import pallas as pl
from jax.experimental.pallas import tpu as pltpu
from jax.experimental.pallas import tpu_sc as plsc

SEQ = 4096
D = 1024
NC = 2
NS = 16
NW = NC * NS
ROWS_PER_W = SEQ // NW
CHUNKS = (8, 24, 32, 32, 24, 8)
CHMAX = max(CHUNKS)
NCHUNK = len(CHUNKS)
OFFS = [sum(CHUNKS[:i]) for i in range(NCHUNK)]
NBUF = 3

_mesh = plsc.VectorSubcoreMesh(core_axis_name="c", subcore_axis_name="s")


@functools.partial(
    pl.kernel,
    mesh=_mesh,
    out_type=jax.ShapeDtypeStruct((1, SEQ, D), jnp.float32),
    scratch_types=(
        [pltpu.VMEM((CHMAX, D), jnp.float32)] * NBUF
        + [pltpu.SemaphoreType.DMA] * (2 * NBUF)
    ),
)
def _slice_copy(enc_hbm, out_hbm, *scratch):
    bufs = scratch[:NBUF]
    in_sems = scratch[NBUF : 2 * NBUF]
    out_sems = scratch[2 * NBUF :]
    wid = lax.axis_index("s") * NC + lax.axis_index("c")
    base = wid * ROWS_PER_W

    in_copies = [None] * NCHUNK
    out_copies = [None] * NCHUNK

    def _scatter(i):
        b = i % NBUF
        in_copies[i].wait()
        out_copies[i] = pltpu.async_copy(
            bufs[b].at[pl.ds(0, CHUNKS[i]), :],
            out_hbm.at[0, pl.ds(base + OFFS[i], CHUNKS[i]), :],
            out_sems[b],
        )

    for i in range(NCHUNK):
        b = i % NBUF
        if i >= NBUF:
            out_copies[i - NBUF].wait()
        in_copies[i] = pltpu.async_copy(
            enc_hbm.at[pl.ds(base + OFFS[i], CHUNKS[i]), :],
            bufs[b].at[pl.ds(0, CHUNKS[i]), :],
            in_sems[b],
        )
        if i >= 1:
            _scatter(i - 1)
    _scatter(NCHUNK - 1)
    for i in range(max(0, NCHUNK - NBUF), NCHUNK):
        out_copies[i].wait()


def kernel(x, encoding):
    del x
    return _slice_copy(encoding)


# speedup vs baseline: 1.7638x; 1.7638x over previous
import jax, jax.numpy as jnp
from jax.experimental import pallas as pl

SEQ, D, BLK = 4096, 1024, 256

def _copy_body(enc_ref, out_ref):
    out_ref[0, ...] = enc_ref[...]

def kernel(x, encoding):
    del x
    return pl.pallas_call(
        _copy_body,
        grid=(SEQ // BLK,),
        in_specs=[pl.BlockSpec((BLK, D), lambda i: (i, 0))],
        out_specs=pl.BlockSpec((1, BLK, D), lambda i: (0, i, 0)),
        out_shape=jax.ShapeDtypeStruct((1, SEQ, D), jnp.float32),
    )(encoding)


# speedup vs baseline: 2.3158x; 1.3130x over previous
import jax, jax.numpy as jnp
from jax.experimental import pallas as pl

SEQ, D, BLK = 4096, 1024, 512

def _copy_body(enc_ref, out_ref):
    out_ref[0, ...] = enc_ref[...]

def kernel(x, encoding):
    del x
    return pl.pallas_call(
        _copy_body,
        grid=(SEQ // BLK,),
        in_specs=[pl.BlockSpec((BLK, D), lambda i: (i, 0))],
        out_specs=pl.BlockSpec((1, BLK, D), lambda i: (0, i, 0)),
        out_shape=jax.ShapeDtypeStruct((1, SEQ, D), jnp.float32),
    )(encoding)


# speedup vs baseline: 2.5515x; 1.1018x over previous
import jax, jax.numpy as jnp
from jax.experimental import pallas as pl

SEQ, D, BLK = 4096, 1024, 1024

def _copy_body(enc_ref, out_ref):
    out_ref[0, ...] = enc_ref[...]

def kernel(x, encoding):
    del x
    return pl.pallas_call(
        _copy_body,
        grid=(SEQ // BLK,),
        in_specs=[pl.BlockSpec((BLK, D), lambda i: (i, 0))],
        out_specs=pl.BlockSpec((1, BLK, D), lambda i: (0, i, 0)),
        out_shape=jax.ShapeDtypeStruct((1, SEQ, D), jnp.float32),
    )(encoding)
